# Initial kernel scaffold; baseline (speedup 1.0000x reference)
import jax
import jax.numpy as jnp
from jax.experimental import pallas as pl


def kernel(responses, elapsed_time, lag_time, response_table, time_W, time_b):
    raise NotImplementedError("write your pallas kernel here")


# speedup vs baseline: 11.5578x; 11.5578x over previous
import jax
import jax.numpy as jnp
from jax.experimental import pallas as pl
from jax.experimental.pallas import tpu as pltpu

_MAX_TIME_ELAPSED = 300000.0
_MAX_TIME_LAG = 86400.0


def _body(resp_ref, e_ref, l_ref, tab_ref, w_ref, b_ref, out_ref):
    rf = resp_ref[...].astype(jnp.float32)
    e = jnp.clip(e_ref[...] * (1.0 / _MAX_TIME_ELAPSED), 0.0, 1.0)
    l = jnp.clip(l_ref[...] * (1.0 / _MAX_TIME_LAG), 0.0, 1.0)
    t0 = tab_ref[0, :]
    diff = tab_ref[1, :] - t0
    base = t0 + b_ref[0, :]
    w0 = w_ref[0, :]
    w1 = w_ref[1, :]
    out_ref[...] = (base[None, None, :]
                    + rf[..., None] * diff[None, None, :]
                    + e[..., None] * w0[None, None, :]
                    + l[..., None] * w1[None, None, :])


def kernel(responses, elapsed_time, lag_time, response_table, time_W, time_b):
    B, L = responses.shape
    D = response_table.shape[1]
    BB = 64
    grid = (B // BB,)
    time_b2 = time_b.reshape(1, D)

    row_spec = pl.BlockSpec((BB, L), lambda i: (i, 0))
    rep_spec = pl.BlockSpec((2, D), lambda i: (0, 0))
    b_spec = pl.BlockSpec((1, D), lambda i: (0, 0))

    return pl.pallas_call(
        _body,
        grid=grid,
        in_specs=[row_spec, row_spec, row_spec, rep_spec, rep_spec, b_spec],
        out_specs=pl.BlockSpec((BB, L, D), lambda i: (i, 0, 0)),
        out_shape=jax.ShapeDtypeStruct((B, L, D), jnp.float32),
    )(responses, elapsed_time, lag_time, response_table, time_W, time_b2)
